# baseline (speedup 1.0000x reference)
import functools

import jax
import jax.numpy as jnp
from jax.experimental import pallas as pl
from jax.experimental.pallas import tpu as pltpu

_HIDDEN = 768
_NUM_EXPERTS = 32
_TOP_K = 4
_FF = 2048
_LIMIT = 7.0
_ALPHA = 1.702

_GU_BLK = 1024
_FF_BLK = _GU_BLK // 2
_NF = (2 * _FF) // _GU_BLK


def _moe_kernel(x_ref, wr_ref, br_ref, wgu_ref, bgu_ref, wd_ref, bd_ref,
                out_ref, scores_ref):
    e = pl.program_id(0)
    f = pl.program_id(1)
    T = x_ref.shape[0]

    @pl.when(jnp.logical_and(e == 0, f == 0))
    def _init():
        x = x_ref[...]
        logits = jax.lax.dot_general(
            x, wr_ref[...], (((1,), (0,)), ((), ())),
            preferred_element_type=jnp.float32) + br_ref[...]
        work = logits
        lane = jax.lax.broadcasted_iota(jnp.int32, logits.shape, 1)
        sel_sum = jnp.zeros_like(logits)
        denom = jnp.zeros((T, 1), jnp.float32)
        top1 = jnp.max(work, axis=-1, keepdims=True)
        for _ in range(_TOP_K):
            m = jnp.max(work, axis=-1, keepdims=True)
            is_max = work == m
            arg = jnp.min(jnp.where(is_max, lane, _NUM_EXPERTS),
                          axis=-1, keepdims=True)
            first = lane == arg
            ev = jnp.exp(m - top1)
            sel_sum = sel_sum + jnp.where(first, ev, 0.0)
            denom = denom + ev
            work = jnp.where(first, -jnp.inf, work)
        scores_ref[...] = sel_sum / denom
        out_ref[...] = jnp.zeros_like(out_ref)

    onehot = (jax.lax.broadcasted_iota(jnp.int32, (_NUM_EXPERTS, 1), 0)
              == e).astype(jnp.float32)
    w_col = jax.lax.dot_general(
        scores_ref[...], onehot, (((1,), (0,)), ((), ())),
        preferred_element_type=jnp.float32)

    x = x_ref[...]
    gu = jax.lax.dot_general(
        x, wgu_ref[...], (((1,), (0,)), ((), ())),
        preferred_element_type=jnp.float32)
    gu = gu + bgu_ref[pl.ds(e, 1), pl.ds(f * _GU_BLK, _GU_BLK)]
    gu3 = gu.reshape(T, _FF_BLK, 2)
    gate = jnp.minimum(gu3[:, :, 0], _LIMIT)
    up = jnp.clip(gu3[:, :, 1], -_LIMIT, _LIMIT)
    glu = gate * jax.nn.sigmoid(gate * _ALPHA)
    h = (up + 1.0) * glu
    y = jax.lax.dot_general(
        h, wd_ref[...], (((1,), (0,)), ((), ())),
        preferred_element_type=jnp.float32)

    @pl.when(f == 0)
    def _bias():
        out_ref[...] += bd_ref[pl.ds(e, 1), :] * w_col

    out_ref[...] += y * w_col


def kernel(hidden_states, Wr, br, W_gate_up, b_gate_up, W_down, b_down):
    B, S, D = hidden_states.shape
    x = hidden_states.reshape(-1, D)
    T = x.shape[0]
    br2 = br.reshape(1, _NUM_EXPERTS)

    out = pl.pallas_call(
        _moe_kernel,
        grid=(_NUM_EXPERTS, _NF),
        in_specs=[
            pl.BlockSpec((T, D), lambda e, f: (0, 0)),
            pl.BlockSpec((D, _NUM_EXPERTS), lambda e, f: (0, 0)),
            pl.BlockSpec((1, _NUM_EXPERTS), lambda e, f: (0, 0)),
            pl.BlockSpec((None, D, _GU_BLK), lambda e, f: (e, 0, f)),
            pl.BlockSpec((_NUM_EXPERTS, 2 * _FF), lambda e, f: (0, 0)),
            pl.BlockSpec((None, _FF_BLK, D), lambda e, f: (e, f, 0)),
            pl.BlockSpec((_NUM_EXPERTS, D), lambda e, f: (0, 0)),
        ],
        out_specs=pl.BlockSpec((T, D), lambda e, f: (0, 0)),
        out_shape=jax.ShapeDtypeStruct((T, D), jnp.float32),
        scratch_shapes=[pltpu.VMEM((T, _NUM_EXPERTS), jnp.float32)],
        compiler_params=pltpu.CompilerParams(
            dimension_semantics=("arbitrary", "arbitrary")),
    )(x, Wr, br2, W_gate_up, b_gate_up, W_down, b_down)
    return out.reshape(B, S, D)


# speedup vs baseline: 7.1996x; 7.1996x over previous
import functools

import jax
import jax.numpy as jnp
from jax.experimental import pallas as pl
from jax.experimental.pallas import tpu as pltpu

_HIDDEN = 768
_NUM_EXPERTS = 32
_TOP_K = 4
_FF = 2048
_LIMIT = 7.0
_ALPHA = 1.702

_GU_BLK = 1024
_FF_BLK = _GU_BLK // 2
_NF = (2 * _FF) // _GU_BLK


def _moe_kernel(x_ref, wr_ref, br_ref, wgu_ref, bgu_ref, wd_ref, bd_ref,
                out_ref, scores_ref, sel_even_ref, sel_odd_ref):
    e = pl.program_id(0)
    f = pl.program_id(1)
    T = x_ref.shape[0]

    @pl.when(jnp.logical_and(e == 0, f == 0))
    def _init():
        r = jax.lax.broadcasted_iota(jnp.int32, (_GU_BLK, _FF_BLK), 0)
        c = jax.lax.broadcasted_iota(jnp.int32, (_GU_BLK, _FF_BLK), 1)
        sel_even_ref[...] = (r == 2 * c).astype(jnp.float32)
        sel_odd_ref[...] = (r == 2 * c + 1).astype(jnp.float32)
        x = x_ref[...]
        logits = jax.lax.dot_general(
            x, wr_ref[...], (((1,), (0,)), ((), ())),
            preferred_element_type=jnp.float32) + br_ref[...]
        work = logits
        lane = jax.lax.broadcasted_iota(jnp.int32, logits.shape, 1)
        sel_sum = jnp.zeros_like(logits)
        denom = jnp.zeros((T, 1), jnp.float32)
        top1 = jnp.max(work, axis=-1, keepdims=True)
        for _ in range(_TOP_K):
            m = jnp.max(work, axis=-1, keepdims=True)
            is_max = work == m
            arg = jnp.min(jnp.where(is_max, lane, _NUM_EXPERTS),
                          axis=-1, keepdims=True)
            first = lane == arg
            ev = jnp.exp(m - top1)
            sel_sum = sel_sum + jnp.where(first, ev, 0.0)
            denom = denom + ev
            work = jnp.where(first, -jnp.inf, work)
        scores_ref[...] = sel_sum / denom
        out_ref[...] = jnp.zeros_like(out_ref)

    onehot = (jax.lax.broadcasted_iota(jnp.int32, (_NUM_EXPERTS, 1), 0)
              == e).astype(jnp.float32)
    w_col = jax.lax.dot_general(
        scores_ref[...], onehot, (((1,), (0,)), ((), ())),
        preferred_element_type=jnp.float32)

    x = x_ref[...]
    gu = jax.lax.dot_general(
        x, wgu_ref[...], (((1,), (0,)), ((), ())),
        preferred_element_type=jnp.float32)
    gu = gu + bgu_ref[pl.ds(e, 1), pl.ds(f * _GU_BLK, _GU_BLK)]
    g = jax.lax.dot_general(
        gu, sel_even_ref[...], (((1,), (0,)), ((), ())),
        preferred_element_type=jnp.float32)
    u = jax.lax.dot_general(
        gu, sel_odd_ref[...], (((1,), (0,)), ((), ())),
        preferred_element_type=jnp.float32)
    gate = jnp.minimum(g, _LIMIT)
    up = jnp.clip(u, -_LIMIT, _LIMIT)
    glu = gate * jax.nn.sigmoid(gate * _ALPHA)
    h = (up + 1.0) * glu
    y = jax.lax.dot_general(
        h, wd_ref[...], (((1,), (0,)), ((), ())),
        preferred_element_type=jnp.float32)

    @pl.when(f == 0)
    def _bias():
        out_ref[...] += bd_ref[pl.ds(e, 1), :] * w_col

    out_ref[...] += y * w_col


def kernel(hidden_states, Wr, br, W_gate_up, b_gate_up, W_down, b_down):
    B, S, D = hidden_states.shape
    x = hidden_states.reshape(-1, D)
    T = x.shape[0]
    br2 = br.reshape(1, _NUM_EXPERTS)

    out = pl.pallas_call(
        _moe_kernel,
        grid=(_NUM_EXPERTS, _NF),
        in_specs=[
            pl.BlockSpec((T, D), lambda e, f: (0, 0)),
            pl.BlockSpec((D, _NUM_EXPERTS), lambda e, f: (0, 0)),
            pl.BlockSpec((1, _NUM_EXPERTS), lambda e, f: (0, 0)),
            pl.BlockSpec((None, D, _GU_BLK), lambda e, f: (e, 0, f)),
            pl.BlockSpec((_NUM_EXPERTS, 2 * _FF), lambda e, f: (0, 0)),
            pl.BlockSpec((None, _FF_BLK, D), lambda e, f: (e, f, 0)),
            pl.BlockSpec((_NUM_EXPERTS, D), lambda e, f: (0, 0)),
        ],
        out_specs=pl.BlockSpec((T, D), lambda e, f: (0, 0)),
        out_shape=jax.ShapeDtypeStruct((T, D), jnp.float32),
        scratch_shapes=[pltpu.VMEM((T, _NUM_EXPERTS), jnp.float32),
                        pltpu.VMEM((_GU_BLK, _FF_BLK), jnp.float32),
                        pltpu.VMEM((_GU_BLK, _FF_BLK), jnp.float32)],
        compiler_params=pltpu.CompilerParams(
            dimension_semantics=("arbitrary", "arbitrary")),
    )(x, Wr, br2, W_gate_up, b_gate_up, W_down, b_down)
    return out.reshape(B, S, D)


# speedup vs baseline: 8.8653x; 1.2314x over previous
import functools

import jax
import jax.numpy as jnp
from jax.experimental import pallas as pl
from jax.experimental.pallas import tpu as pltpu

_HIDDEN = 768
_NUM_EXPERTS = 32
_TOP_K = 4
_FF = 2048
_LIMIT = 7.0
_ALPHA = 1.702

_GU_BLK = 2048
_FF_BLK = _GU_BLK // 2
_NF = (2 * _FF) // _GU_BLK
_SEL = 1024
_NSPLIT = _GU_BLK // _SEL


def _moe_kernel(x_ref, wr_ref, br_ref, wgu_ref, bgu_ref, wd_ref, bd_ref,
                out_ref, scores_ref, sel_even_ref, sel_odd_ref):
    e = pl.program_id(0)
    f = pl.program_id(1)
    T = x_ref.shape[0]

    @pl.when(jnp.logical_and(e == 0, f == 0))
    def _init():
        r = jax.lax.broadcasted_iota(jnp.int32, (_SEL, _SEL // 2), 0)
        c = jax.lax.broadcasted_iota(jnp.int32, (_SEL, _SEL // 2), 1)
        sel_even_ref[...] = (r == 2 * c).astype(jnp.float32)
        sel_odd_ref[...] = (r == 2 * c + 1).astype(jnp.float32)
        x = x_ref[...]
        logits = jax.lax.dot_general(
            x, wr_ref[...], (((1,), (0,)), ((), ())),
            preferred_element_type=jnp.float32) + br_ref[...]
        work = logits
        lane = jax.lax.broadcasted_iota(jnp.int32, logits.shape, 1)
        sel_sum = jnp.zeros_like(logits)
        denom = jnp.zeros((T, 1), jnp.float32)
        top1 = jnp.max(work, axis=-1, keepdims=True)
        for _ in range(_TOP_K):
            m = jnp.max(work, axis=-1, keepdims=True)
            is_max = work == m
            arg = jnp.min(jnp.where(is_max, lane, _NUM_EXPERTS),
                          axis=-1, keepdims=True)
            first = lane == arg
            ev = jnp.exp(m - top1)
            sel_sum = sel_sum + jnp.where(first, ev, 0.0)
            denom = denom + ev
            work = jnp.where(first, -jnp.inf, work)
        scores_ref[...] = sel_sum / denom
        out_ref[...] = jnp.zeros_like(out_ref)

    onehot = (jax.lax.broadcasted_iota(jnp.int32, (_NUM_EXPERTS, 1), 0)
              == e).astype(jnp.float32)
    w_col = jax.lax.dot_general(
        scores_ref[...], onehot, (((1,), (0,)), ((), ())),
        preferred_element_type=jnp.float32)

    x = x_ref[...]
    gu = jax.lax.dot_general(
        x, wgu_ref[...], (((1,), (0,)), ((), ())),
        preferred_element_type=jnp.float32)
    gu = gu + bgu_ref[pl.ds(e, 1), pl.ds(f * _GU_BLK, _GU_BLK)]
    se = sel_even_ref[...]
    so = sel_odd_ref[...]
    g_parts = []
    u_parts = []
    for s in range(_NSPLIT):
        gu_s = gu[:, s * _SEL:(s + 1) * _SEL]
        g_parts.append(jax.lax.dot_general(
            gu_s, se, (((1,), (0,)), ((), ())),
            preferred_element_type=jnp.float32))
        u_parts.append(jax.lax.dot_general(
            gu_s, so, (((1,), (0,)), ((), ())),
            preferred_element_type=jnp.float32))
    g = jnp.concatenate(g_parts, axis=1) if _NSPLIT > 1 else g_parts[0]
    u = jnp.concatenate(u_parts, axis=1) if _NSPLIT > 1 else u_parts[0]
    gate = jnp.minimum(g, _LIMIT)
    up = jnp.clip(u, -_LIMIT, _LIMIT)
    glu = gate * jax.nn.sigmoid(gate * _ALPHA)
    h = (up + 1.0) * glu
    y = jax.lax.dot_general(
        h, wd_ref[...], (((1,), (0,)), ((), ())),
        preferred_element_type=jnp.float32)

    @pl.when(f == 0)
    def _bias():
        out_ref[...] += bd_ref[pl.ds(e, 1), :] * w_col

    out_ref[...] += y * w_col


def kernel(hidden_states, Wr, br, W_gate_up, b_gate_up, W_down, b_down):
    B, S, D = hidden_states.shape
    x = hidden_states.reshape(-1, D)
    T = x.shape[0]
    br2 = br.reshape(1, _NUM_EXPERTS)

    out = pl.pallas_call(
        _moe_kernel,
        grid=(_NUM_EXPERTS, _NF),
        in_specs=[
            pl.BlockSpec((T, D), lambda e, f: (0, 0)),
            pl.BlockSpec((D, _NUM_EXPERTS), lambda e, f: (0, 0)),
            pl.BlockSpec((1, _NUM_EXPERTS), lambda e, f: (0, 0)),
            pl.BlockSpec((None, D, _GU_BLK), lambda e, f: (e, 0, f)),
            pl.BlockSpec((_NUM_EXPERTS, 2 * _FF), lambda e, f: (0, 0)),
            pl.BlockSpec((None, _FF_BLK, D), lambda e, f: (e, f, 0)),
            pl.BlockSpec((_NUM_EXPERTS, D), lambda e, f: (0, 0)),
        ],
        out_specs=pl.BlockSpec((T, D), lambda e, f: (0, 0)),
        out_shape=jax.ShapeDtypeStruct((T, D), jnp.float32),
        scratch_shapes=[pltpu.VMEM((T, _NUM_EXPERTS), jnp.float32),
                        pltpu.VMEM((_SEL, _SEL // 2), jnp.float32),
                        pltpu.VMEM((_SEL, _SEL // 2), jnp.float32)],
        compiler_params=pltpu.CompilerParams(
            dimension_semantics=("arbitrary", "arbitrary")),
    )(x, Wr, br2, W_gate_up, b_gate_up, W_down, b_down)
    return out.reshape(B, S, D)


# speedup vs baseline: 8.9638x; 1.0111x over previous
import functools

import jax
import jax.numpy as jnp
from jax.experimental import pallas as pl
from jax.experimental.pallas import tpu as pltpu

_HIDDEN = 768
_NUM_EXPERTS = 32
_TOP_K = 4
_FF = 2048
_LIMIT = 7.0
_ALPHA = 1.702

_GU_BLK = 4096
_FF_BLK = _GU_BLK // 2
_NF = (2 * _FF) // _GU_BLK
_SEL = 1024
_NSPLIT = _GU_BLK // _SEL


def _moe_kernel(x_ref, wr_ref, br_ref, wgu_ref, bgu_ref, wd_ref, bd_ref,
                out_ref, scores_ref, sel_even_ref, sel_odd_ref):
    e = pl.program_id(0)
    f = pl.program_id(1)
    T = x_ref.shape[0]

    @pl.when(jnp.logical_and(e == 0, f == 0))
    def _init():
        r = jax.lax.broadcasted_iota(jnp.int32, (_SEL, _SEL // 2), 0)
        c = jax.lax.broadcasted_iota(jnp.int32, (_SEL, _SEL // 2), 1)
        sel_even_ref[...] = (r == 2 * c).astype(jnp.float32)
        sel_odd_ref[...] = (r == 2 * c + 1).astype(jnp.float32)
        x = x_ref[...]
        logits = jax.lax.dot_general(
            x, wr_ref[...], (((1,), (0,)), ((), ())),
            preferred_element_type=jnp.float32) + br_ref[...]
        work = logits
        lane = jax.lax.broadcasted_iota(jnp.int32, logits.shape, 1)
        sel_sum = jnp.zeros_like(logits)
        denom = jnp.zeros((T, 1), jnp.float32)
        top1 = jnp.max(work, axis=-1, keepdims=True)
        for _ in range(_TOP_K):
            m = jnp.max(work, axis=-1, keepdims=True)
            is_max = work == m
            arg = jnp.min(jnp.where(is_max, lane, _NUM_EXPERTS),
                          axis=-1, keepdims=True)
            first = lane == arg
            ev = jnp.exp(m - top1)
            sel_sum = sel_sum + jnp.where(first, ev, 0.0)
            denom = denom + ev
            work = jnp.where(first, -jnp.inf, work)
        scores_ref[...] = sel_sum / denom
        out_ref[...] = jnp.zeros_like(out_ref)

    onehot = (jax.lax.broadcasted_iota(jnp.int32, (_NUM_EXPERTS, 1), 0)
              == e).astype(jnp.float32)
    w_col = jax.lax.dot_general(
        scores_ref[...], onehot, (((1,), (0,)), ((), ())),
        preferred_element_type=jnp.float32)

    x = x_ref[...]
    gu = jax.lax.dot_general(
        x, wgu_ref[...], (((1,), (0,)), ((), ())),
        preferred_element_type=jnp.float32)
    gu = gu + bgu_ref[pl.ds(e, 1), pl.ds(f * _GU_BLK, _GU_BLK)]
    se = sel_even_ref[...]
    so = sel_odd_ref[...]
    g_parts = []
    u_parts = []
    for s in range(_NSPLIT):
        gu_s = gu[:, s * _SEL:(s + 1) * _SEL]
        g_parts.append(jax.lax.dot_general(
            gu_s, se, (((1,), (0,)), ((), ())),
            preferred_element_type=jnp.float32))
        u_parts.append(jax.lax.dot_general(
            gu_s, so, (((1,), (0,)), ((), ())),
            preferred_element_type=jnp.float32))
    g = jnp.concatenate(g_parts, axis=1) if _NSPLIT > 1 else g_parts[0]
    u = jnp.concatenate(u_parts, axis=1) if _NSPLIT > 1 else u_parts[0]
    gate = jnp.minimum(g, _LIMIT)
    up = jnp.clip(u, -_LIMIT, _LIMIT)
    glu = gate * jax.nn.sigmoid(gate * _ALPHA)
    h = (up + 1.0) * glu
    y = jax.lax.dot_general(
        h, wd_ref[...], (((1,), (0,)), ((), ())),
        preferred_element_type=jnp.float32)

    @pl.when(f == 0)
    def _bias():
        out_ref[...] += bd_ref[pl.ds(e, 1), :] * w_col

    out_ref[...] += y * w_col


def kernel(hidden_states, Wr, br, W_gate_up, b_gate_up, W_down, b_down):
    B, S, D = hidden_states.shape
    x = hidden_states.reshape(-1, D)
    T = x.shape[0]
    br2 = br.reshape(1, _NUM_EXPERTS)

    out = pl.pallas_call(
        _moe_kernel,
        grid=(_NUM_EXPERTS, _NF),
        in_specs=[
            pl.BlockSpec((T, D), lambda e, f: (0, 0)),
            pl.BlockSpec((D, _NUM_EXPERTS), lambda e, f: (0, 0)),
            pl.BlockSpec((1, _NUM_EXPERTS), lambda e, f: (0, 0)),
            pl.BlockSpec((None, D, _GU_BLK), lambda e, f: (e, 0, f)),
            pl.BlockSpec((_NUM_EXPERTS, 2 * _FF), lambda e, f: (0, 0)),
            pl.BlockSpec((None, _FF_BLK, D), lambda e, f: (e, f, 0)),
            pl.BlockSpec((_NUM_EXPERTS, D), lambda e, f: (0, 0)),
        ],
        out_specs=pl.BlockSpec((T, D), lambda e, f: (0, 0)),
        out_shape=jax.ShapeDtypeStruct((T, D), jnp.float32),
        scratch_shapes=[pltpu.VMEM((T, _NUM_EXPERTS), jnp.float32),
                        pltpu.VMEM((_SEL, _SEL // 2), jnp.float32),
                        pltpu.VMEM((_SEL, _SEL // 2), jnp.float32)],
        compiler_params=pltpu.CompilerParams(
            dimension_semantics=("arbitrary", "arbitrary")),
    )(x, Wr, br2, W_gate_up, b_gate_up, W_down, b_down)
    return out.reshape(B, S, D)


# speedup vs baseline: 9.0554x; 1.0102x over previous
import jax
import jax.numpy as jnp
from jax.experimental import pallas as pl
from jax.experimental.pallas import tpu as pltpu

_HIDDEN = 768
_NUM_EXPERTS = 32
_TOP_K = 4
_FF = 2048
_LIMIT = 7.0
_ALPHA = 1.702

_NCORES = 2
_EPC = _NUM_EXPERTS // _NCORES
_GU_BLK = 4096
_FF_BLK = _GU_BLK // 2
_NF = (2 * _FF) // _GU_BLK
_SEL = 1024
_NSPLIT = _GU_BLK // _SEL


def _moe_kernel(x_ref, wr_ref, br_ref, wgu_ref, bgu_ref, wd_ref, bd_ref,
                out_ref, scores_ref, sel_even_ref, sel_odd_ref):
    c = pl.program_id(0)
    el = pl.program_id(1)
    f = pl.program_id(2)
    e = c * _EPC + el
    T = x_ref.shape[0]

    @pl.when(jnp.logical_and(el == 0, f == 0))
    def _init():
        r = jax.lax.broadcasted_iota(jnp.int32, (_SEL, _SEL // 2), 0)
        cc = jax.lax.broadcasted_iota(jnp.int32, (_SEL, _SEL // 2), 1)
        sel_even_ref[...] = (r == 2 * cc).astype(jnp.float32)
        sel_odd_ref[...] = (r == 2 * cc + 1).astype(jnp.float32)
        x = x_ref[...]
        logits = jax.lax.dot_general(
            x, wr_ref[...], (((1,), (0,)), ((), ())),
            preferred_element_type=jnp.float32) + br_ref[...]
        work = logits
        lane = jax.lax.broadcasted_iota(jnp.int32, logits.shape, 1)
        sel_sum = jnp.zeros_like(logits)
        denom = jnp.zeros((T, 1), jnp.float32)
        top1 = jnp.max(work, axis=-1, keepdims=True)
        for _ in range(_TOP_K):
            m = jnp.max(work, axis=-1, keepdims=True)
            is_max = work == m
            arg = jnp.min(jnp.where(is_max, lane, _NUM_EXPERTS),
                          axis=-1, keepdims=True)
            first = lane == arg
            ev = jnp.exp(m - top1)
            sel_sum = sel_sum + jnp.where(first, ev, 0.0)
            denom = denom + ev
            work = jnp.where(first, -jnp.inf, work)
        scores_ref[...] = sel_sum / denom
        out_ref[...] = jnp.zeros_like(out_ref)

    onehot = (jax.lax.broadcasted_iota(jnp.int32, (_NUM_EXPERTS, 1), 0)
              == e).astype(jnp.float32)
    w_col = jax.lax.dot_general(
        scores_ref[...], onehot, (((1,), (0,)), ((), ())),
        preferred_element_type=jnp.float32)

    x = x_ref[...]
    gu = jax.lax.dot_general(
        x, wgu_ref[...], (((1,), (0,)), ((), ())),
        preferred_element_type=jnp.float32)
    gu = gu + bgu_ref[pl.ds(e, 1), pl.ds(f * _GU_BLK, _GU_BLK)]
    se = sel_even_ref[...]
    so = sel_odd_ref[...]
    g_parts = []
    u_parts = []
    for s in range(_NSPLIT):
        gu_s = gu[:, s * _SEL:(s + 1) * _SEL]
        g_parts.append(jax.lax.dot_general(
            gu_s, se, (((1,), (0,)), ((), ())),
            preferred_element_type=jnp.float32))
        u_parts.append(jax.lax.dot_general(
            gu_s, so, (((1,), (0,)), ((), ())),
            preferred_element_type=jnp.float32))
    g = jnp.concatenate(g_parts, axis=1) if _NSPLIT > 1 else g_parts[0]
    u = jnp.concatenate(u_parts, axis=1) if _NSPLIT > 1 else u_parts[0]
    gate = jnp.minimum(g, _LIMIT)
    up = jnp.clip(u, -_LIMIT, _LIMIT)
    glu = gate * jax.nn.sigmoid(gate * _ALPHA)
    h = (up + 1.0) * glu
    y = jax.lax.dot_general(
        h, wd_ref[...], (((1,), (0,)), ((), ())),
        preferred_element_type=jnp.float32)

    @pl.when(f == 0)
    def _bias():
        out_ref[...] += bd_ref[pl.ds(e, 1), :] * w_col

    out_ref[...] += y * w_col


def kernel(hidden_states, Wr, br, W_gate_up, b_gate_up, W_down, b_down):
    B, S, D = hidden_states.shape
    x = hidden_states.reshape(-1, D)
    T = x.shape[0]
    br2 = br.reshape(1, _NUM_EXPERTS)

    parts = pl.pallas_call(
        _moe_kernel,
        grid=(_NCORES, _EPC, _NF),
        in_specs=[
            pl.BlockSpec((T, D), lambda c, e, f: (0, 0)),
            pl.BlockSpec((D, _NUM_EXPERTS), lambda c, e, f: (0, 0)),
            pl.BlockSpec((1, _NUM_EXPERTS), lambda c, e, f: (0, 0)),
            pl.BlockSpec((None, D, _GU_BLK),
                         lambda c, e, f: (c * _EPC + e, 0, f)),
            pl.BlockSpec((_NUM_EXPERTS, 2 * _FF),
                         lambda c, e, f: (0, 0)),
            pl.BlockSpec((None, _FF_BLK, D),
                         lambda c, e, f: (c * _EPC + e, f, 0)),
            pl.BlockSpec((_NUM_EXPERTS, D), lambda c, e, f: (0, 0)),
        ],
        out_specs=pl.BlockSpec((None, T, D), lambda c, e, f: (c, 0, 0)),
        out_shape=jax.ShapeDtypeStruct((_NCORES, T, D), jnp.float32),
        scratch_shapes=[pltpu.VMEM((T, _NUM_EXPERTS), jnp.float32),
                        pltpu.VMEM((_SEL, _SEL // 2), jnp.float32),
                        pltpu.VMEM((_SEL, _SEL // 2), jnp.float32)],
        compiler_params=pltpu.CompilerParams(
            dimension_semantics=("parallel", "arbitrary", "arbitrary")),
    )(x, Wr, br2, W_gate_up, b_gate_up, W_down, b_down)
    out = parts[0] + parts[1]
    return out.reshape(B, S, D)
